# baseline (speedup 1.0000x reference)
import functools

import jax
import jax.numpy as jnp
from jax import lax
from jax.experimental import pallas as pl
from jax.experimental.pallas import tpu as pltpu
from jax.experimental.pallas import tpu_sc as plsc

_NC = 2
_NS = 16
_NW = _NC * _NS
_LANES = 16
_CHUNK_P = 16
_DEPAD_BLOCK = 20000


def _depad_body(x_ref, out_ref):
    x = x_ref[...]
    out_ref[...] = jnp.concatenate([x, jnp.zeros_like(x)], axis=-1)


@functools.lru_cache(maxsize=None)
def _build_depad(vocab, dim):
    grid = vocab // _DEPAD_BLOCK
    return pl.pallas_call(
        _depad_body,
        grid=(grid,),
        in_specs=[pl.BlockSpec((_DEPAD_BLOCK, dim), lambda i: (i, 0))],
        out_specs=pl.BlockSpec((_DEPAD_BLOCK, 2 * dim), lambda i: (i, 0)),
        out_shape=jax.ShapeDtypeStruct((vocab, 2 * dim), jnp.float32),
    )


@functools.lru_cache(maxsize=None)
def _build_sc_partials(n_phrases, l_u, l_v, dim, vocab):
    phr_t = n_phrases // _NW
    nch = phr_t // _CHUNK_P
    rows_ch = _CHUNK_P * l_v
    scale = 1.0 / float(l_u * l_v)
    nc = dim // _LANES
    srow = phr_t * _LANES // 128

    @functools.partial(
        pl.kernel,
        mesh=plsc.VectorSubcoreMesh(core_axis_name="c", subcore_axis_name="s"),
        out_type=jax.ShapeDtypeStruct((_NW, srow, 128), jnp.float32),
        scratch_types=[
            pltpu.VMEM((l_u, 2 * dim), jnp.float32),
            pltpu.VMEM((nch, rows_ch), jnp.int32),
            pltpu.VMEM((2, rows_ch, 2 * dim), jnp.float32),
            pltpu.VMEM((srow, 128), jnp.float32),
            pltpu.SemaphoreType.DMA,
            pltpu.SemaphoreType.DMA,
        ],
    )
    def sc_partials(u_rows_hbm, idx_hbm, v_w_hbm, out_hbm,
                    u_rows_v, idx_v, rows_v, scores_v, sem0, sem1):
        wid = lax.axis_index("s") * _NC + lax.axis_index("c")
        sems = (sem0, sem1)

        pltpu.sync_copy(u_rows_hbm, u_rows_v)
        su = []
        for c in range(nc):
            s = u_rows_v[0, pl.ds(c * _LANES, _LANES)]
            for j in range(1, l_u):
                s = s + u_rows_v[j, pl.ds(c * _LANES, _LANES)]
            su.append(s * scale)

        pltpu.sync_copy(idx_hbm.at[wid], idx_v)

        pltpu.async_copy(v_w_hbm.at[idx_v.at[0]], rows_v.at[0], sem0)

        def outer(g, carry):
            for b in range(2):
                j = g * 2 + b

                @pl.when(j + 1 < nch)
                def _():
                    pltpu.async_copy(
                        v_w_hbm.at[idx_v.at[j + 1]], rows_v.at[1 - b], sems[1 - b])

                pltpu.make_async_copy(
                    v_w_hbm.at[idx_v.at[j]], rows_v.at[b], sems[b]).wait()

                for p in range(_CHUNK_P):
                    acc = None
                    for c in range(nc):
                        s = rows_v[b, p * l_v, pl.ds(c * _LANES, _LANES)]
                        for l in range(1, l_v):
                            s = s + rows_v[b, p * l_v + l, pl.ds(c * _LANES, _LANES)]
                        term = s * su[c]
                        acc = term if acc is None else acc + term
                    scores_v[2 * j + p // 8, pl.ds((p % 8) * _LANES, _LANES)] = acc
            return carry

        lax.fori_loop(0, nch // 2, outer, None)
        pltpu.sync_copy(scores_v, out_hbm.at[wid])

    return sc_partials


_PHR_PER_ROW = 128 // _LANES


def _tc_loss_body(n_pos, x_ref, out_ref):
    x = x_ref[...]
    lane_grp = lax.broadcasted_iota(jnp.int32, (128, _PHR_PER_ROW), 0) // _LANES
    col = lax.broadcasted_iota(jnp.int32, (128, _PHR_PER_ROW), 1)
    sel = (lane_grp == col).astype(jnp.float32)
    score = jnp.dot(x, sel, preferred_element_type=jnp.float32)
    rows = score.shape[0]
    pid = (lax.broadcasted_iota(jnp.int32, (rows, _PHR_PER_ROW), 0)
           * _PHR_PER_ROW
           + lax.broadcasted_iota(jnp.int32, (rows, _PHR_PER_ROW), 1))
    z = jnp.where(pid < n_pos, -score, score)
    sp = jnp.maximum(z, 0.0) + jnp.log(1.0 + jnp.exp(-jnp.abs(z)))
    out_ref[0, 0] = jnp.sum(sp)


def kernel(pos_u, pos_v, neg_v, u_weight, v_weight):
    n_pos, l_v = pos_v.shape
    n_neg = neg_v.shape[0]
    l_u = pos_u.shape[0]
    vocab, dim = u_weight.shape
    n_phr = n_pos + n_neg

    idx = jnp.concatenate(
        [pos_v.reshape(-1), neg_v.reshape(-1)]).astype(jnp.int32)
    idx = idx.reshape(_NW, (n_phr // _NW) // _CHUNK_P, _CHUNK_P * l_v)

    u_rows = jnp.pad(
        jnp.take(u_weight, pos_u.astype(jnp.int32), axis=0), ((0, 0), (0, dim)))

    v128 = _build_depad(vocab, dim)(v_weight)

    sc_partials = _build_sc_partials(n_phr, l_u, l_v, dim, vocab)
    partials = sc_partials(u_rows, idx, v128)

    loss = pl.pallas_call(
        functools.partial(_tc_loss_body, n_pos),
        out_shape=jax.ShapeDtypeStruct((1, 1), jnp.float32),
        out_specs=pl.BlockSpec(memory_space=pltpu.SMEM),
    )(partials.reshape(n_phr // _PHR_PER_ROW, 128))
    return loss[0, 0]


# speedup vs baseline: 1.5598x; 1.5598x over previous
import functools

import jax
import jax.numpy as jnp
from jax import lax
from jax.experimental import pallas as pl
from jax.experimental.pallas import tpu as pltpu
from jax.experimental.pallas import tpu_sc as plsc

_NC = 2
_NS = 16
_NW = _NC * _NS
_LANES = 16
_CHUNK_P = 16
_DEPAD_BLOCK = 4096


def _depad_body(x_ref, out_ref):
    x = x_ref[...]
    y = x.T
    out_ref[...] = jnp.concatenate([y, jnp.zeros_like(y)], axis=-1)


@functools.lru_cache(maxsize=None)
def _build_depad(vocab, dim):
    grid = pl.cdiv(vocab, _DEPAD_BLOCK)
    return pl.pallas_call(
        _depad_body,
        grid=(grid,),
        in_specs=[pl.BlockSpec((dim, _DEPAD_BLOCK), lambda i: (0, i))],
        out_specs=pl.BlockSpec((_DEPAD_BLOCK, 2 * dim), lambda i: (i, 0)),
        out_shape=jax.ShapeDtypeStruct((vocab, 2 * dim), jnp.float32),
    )


@functools.lru_cache(maxsize=None)
def _build_sc_partials(n_phrases, l_u, l_v, dim, vocab):
    phr_t = n_phrases // _NW
    nch = phr_t // _CHUNK_P
    rows_ch = _CHUNK_P * l_v
    scale = 1.0 / float(l_u * l_v)
    nc = dim // _LANES
    srow = phr_t * _LANES // 128

    @functools.partial(
        pl.kernel,
        mesh=plsc.VectorSubcoreMesh(core_axis_name="c", subcore_axis_name="s"),
        out_type=jax.ShapeDtypeStruct((_NW, srow, 128), jnp.float32),
        scratch_types=[
            pltpu.VMEM((l_u, 2 * dim), jnp.float32),
            pltpu.VMEM((nch, rows_ch), jnp.int32),
            pltpu.VMEM((2, rows_ch, 2 * dim), jnp.float32),
            pltpu.VMEM((srow, 128), jnp.float32),
            pltpu.SemaphoreType.DMA,
            pltpu.SemaphoreType.DMA,
        ],
    )
    def sc_partials(u_rows_hbm, idx_hbm, v_w_hbm, out_hbm,
                    u_rows_v, idx_v, rows_v, scores_v, sem0, sem1):
        wid = lax.axis_index("s") * _NC + lax.axis_index("c")
        sems = (sem0, sem1)

        pltpu.sync_copy(u_rows_hbm, u_rows_v)
        su = []
        for c in range(nc):
            s = u_rows_v[0, pl.ds(c * _LANES, _LANES)]
            for j in range(1, l_u):
                s = s + u_rows_v[j, pl.ds(c * _LANES, _LANES)]
            su.append(s * scale)

        pltpu.sync_copy(idx_hbm.at[wid], idx_v)

        pltpu.async_copy(v_w_hbm.at[idx_v.at[0]], rows_v.at[0], sem0)

        def outer(g, carry):
            for b in range(2):
                j = g * 2 + b

                @pl.when(j + 1 < nch)
                def _():
                    pltpu.async_copy(
                        v_w_hbm.at[idx_v.at[j + 1]], rows_v.at[1 - b], sems[1 - b])

                pltpu.make_async_copy(
                    v_w_hbm.at[idx_v.at[j]], rows_v.at[b], sems[b]).wait()

                for p in range(_CHUNK_P):
                    acc = None
                    for c in range(nc):
                        s = rows_v[b, p * l_v, pl.ds(c * _LANES, _LANES)]
                        for l in range(1, l_v):
                            s = s + rows_v[b, p * l_v + l, pl.ds(c * _LANES, _LANES)]
                        term = s * su[c]
                        acc = term if acc is None else acc + term
                    scores_v[2 * j + p // 8, pl.ds((p % 8) * _LANES, _LANES)] = acc
            return carry

        lax.fori_loop(0, nch // 2, outer, None)
        pltpu.sync_copy(scores_v, out_hbm.at[wid])

    return sc_partials


_PHR_PER_ROW = 128 // _LANES


def _tc_loss_body(n_pos, x_ref, out_ref):
    x = x_ref[...]
    lane_grp = lax.broadcasted_iota(jnp.int32, (128, _PHR_PER_ROW), 0) // _LANES
    col = lax.broadcasted_iota(jnp.int32, (128, _PHR_PER_ROW), 1)
    sel = (lane_grp == col).astype(jnp.float32)
    score = jnp.dot(x, sel, preferred_element_type=jnp.float32)
    rows = score.shape[0]
    pid = (lax.broadcasted_iota(jnp.int32, (rows, _PHR_PER_ROW), 0)
           * _PHR_PER_ROW
           + lax.broadcasted_iota(jnp.int32, (rows, _PHR_PER_ROW), 1))
    z = jnp.where(pid < n_pos, -score, score)
    sp = jnp.maximum(z, 0.0) + jnp.log(1.0 + jnp.exp(-jnp.abs(z)))
    out_ref[0, 0] = jnp.sum(sp)


def kernel(pos_u, pos_v, neg_v, u_weight, v_weight):
    n_pos, l_v = pos_v.shape
    n_neg = neg_v.shape[0]
    l_u = pos_u.shape[0]
    vocab, dim = u_weight.shape
    n_phr = n_pos + n_neg

    idx = jnp.concatenate(
        [pos_v.reshape(-1), neg_v.reshape(-1)]).astype(jnp.int32)
    idx = idx.reshape(_NW, (n_phr // _NW) // _CHUNK_P, _CHUNK_P * l_v)

    u_rows = jnp.pad(
        jnp.take(u_weight, pos_u.astype(jnp.int32), axis=0), ((0, 0), (0, dim)))

    v128 = _build_depad(vocab, dim)(v_weight.T)

    sc_partials = _build_sc_partials(n_phr, l_u, l_v, dim, vocab)
    partials = sc_partials(u_rows, idx, v128)

    loss = pl.pallas_call(
        functools.partial(_tc_loss_body, n_pos),
        out_shape=jax.ShapeDtypeStruct((1, 1), jnp.float32),
        out_specs=pl.BlockSpec(memory_space=pltpu.SMEM),
    )(partials.reshape(n_phr // _PHR_PER_ROW, 128))
    return loss[0, 0]
